# baseline (speedup 1.0000x reference)
import functools

import jax
import jax.numpy as jnp
from jax import lax
from jax.experimental import pallas as pl
from jax.experimental.pallas import tpu as pltpu
from jax.experimental.pallas import tpu_sc as plsc

MARGIN = 1.0
N_SAMPLES = 65536
B = 4
E = 32
HW = 512 * 512
BHW = B * HW

NC = 2
NS = 16
NW = NC * NS
PAIRS = B * N_SAMPLES
PPW = PAIRS // NW
CHUNK = 128
ROWS_PW = PPW // CHUNK
NT = ROWS_PW // 2

TRBLK = 2048


def _build_indices():
    i1, i2 = [], []
    for b in range(B):
        kb = jax.random.fold_in(jax.random.key(42), b)
        ka, kc = jax.random.split(kb)
        i1.append(jax.random.randint(ka, (N_SAMPLES,), 0, HW) + b * HW)
        i2.append(jax.random.randint(kc, (N_SAMPLES,), 0, HW) + b * HW)
    idx1 = jnp.concatenate(i1).astype(jnp.int32).reshape(PAIRS // CHUNK, CHUNK)
    idx2 = jnp.concatenate(i2).astype(jnp.int32).reshape(PAIRS // CHUNK, CHUNK)
    return idx1, idx2


def _tr_body(x_ref, o_ref):
    o_ref[0] = x_ref[0].T


def _transpose(emb3):
    return pl.pallas_call(
        _tr_body,
        grid=(B, HW // TRBLK),
        in_specs=[pl.BlockSpec((1, E, TRBLK), lambda b, j: (b, 0, j))],
        out_specs=pl.BlockSpec((1, TRBLK, E), lambda b, j: (b, j, 0)),
        out_shape=jax.ShapeDtypeStruct((B, HW, E), jnp.float32),
    )(emb3)


def _chunk_contrib(a_ref, b_ref, l1_ref, l2_ref, accs):
    pos_s, pos_c, neg_s, neg_c = accs
    lane = lax.iota(jnp.int32, 16)
    one = jnp.float32(1.0)
    zero = jnp.float32(0.0)
    for g in range(CHUNK // 16):
        rows = lane + (g * 16)
        d2 = jnp.zeros(16, jnp.float32)
        for c in range(E):
            cols = jnp.full((16,), c, jnp.int32)
            av = plsc.load_gather(a_ref, [rows, cols])
            bv = plsc.load_gather(b_ref, [rows, cols])
            d = av - bv
            d2 = d2 + d * d
        l1 = l1_ref[pl.ds(g * 16, 16)]
        l2 = l2_ref[pl.ds(g * 16, 16)]
        same = (l1 == l2) & (l1 != 0)
        diff = (l1 != l2) & (l1 != 0) & (l2 != 0)
        d2e = d2 + jnp.float32(1e-12)
        ir = jnp.int32(0x5F3759DF) - (plsc.bitcast(d2e, jnp.int32) >> 1)
        r = plsc.bitcast(ir, jnp.float32)
        r = r * (jnp.float32(1.5) - jnp.float32(0.5) * d2e * r * r)
        r = r * (jnp.float32(1.5) - jnp.float32(0.5) * d2e * r * r)
        dist = d2e * r
        hin = jnp.maximum(jnp.float32(MARGIN) - dist, zero)
        pos_s = pos_s + jnp.where(same, d2e, zero)
        pos_c = pos_c + jnp.where(same, one, zero)
        neg_s = neg_s + jnp.where(diff, hin * hin, zero)
        neg_c = neg_c + jnp.where(diff, one, zero)
    return pos_s, pos_c, neg_s, neg_c


def _sc_body(emb_hbm, lab_hbm, idx1_hbm, idx2_hbm, out_hbm,
             idx1_v, idx2_v, a0, a1, b0, b1, l10, l11, l20, l21,
             accv, sem0, sem1):
    wid = lax.axis_index("s") * NC + lax.axis_index("c")
    base = wid * ROWS_PW
    pltpu.sync_copy(idx1_hbm.at[pl.ds(base, ROWS_PW)], idx1_v)
    pltpu.sync_copy(idx2_hbm.at[pl.ds(base, ROWS_PW)], idx2_v)

    bufs = ((a0, b0, l10, l20, sem0), (a1, b1, l11, l21, sem1))

    def _issue(jj, a, b, l1, l2, sem):
        pltpu.async_copy(emb_hbm.at[idx1_v.at[jj]], a, sem)
        pltpu.async_copy(emb_hbm.at[idx2_v.at[jj]], b, sem)
        pltpu.async_copy(lab_hbm.at[idx1_v.at[jj]], l1, sem)
        pltpu.async_copy(lab_hbm.at[idx2_v.at[jj]], l2, sem)

    def _drain(a, b, l1, l2, sem):
        pltpu.make_async_copy(emb_hbm.at[idx1_v.at[0]], a, sem).wait()
        pltpu.make_async_copy(emb_hbm.at[idx2_v.at[0]], b, sem).wait()
        pltpu.make_async_copy(lab_hbm.at[idx1_v.at[0]], l1, sem).wait()
        pltpu.make_async_copy(lab_hbm.at[idx2_v.at[0]], l2, sem).wait()

    _issue(jnp.int32(0), *bufs[0])

    def body(t, accs):
        j0 = 2 * t
        _issue(j0 + 1, *bufs[1])
        _drain(*bufs[0])
        accs = _chunk_contrib(bufs[0][0], bufs[0][1], bufs[0][2], bufs[0][3],
                              accs)

        @pl.when(t < NT - 1)
        def _():
            _issue(j0 + 2, *bufs[0])

        _drain(*bufs[1])
        accs = _chunk_contrib(bufs[1][0], bufs[1][1], bufs[1][2], bufs[1][3],
                              accs)
        return accs

    z = jnp.zeros(16, jnp.float32)
    pos_s, pos_c, neg_s, neg_c = lax.fori_loop(0, NT, body, (z, z, z, z))
    accv[0] = pos_s
    accv[1] = pos_c
    accv[2] = neg_s
    accv[3] = neg_c
    pltpu.sync_copy(accv, out_hbm.at[wid])


_sc_kernel = functools.partial(
    pl.kernel,
    out_type=jax.ShapeDtypeStruct((NW, 4, 16), jnp.float32),
    mesh=plsc.VectorSubcoreMesh(core_axis_name="c", subcore_axis_name="s",
                                num_cores=NC, num_subcores=NS),
    scratch_types=[
        pltpu.VMEM((ROWS_PW, CHUNK), jnp.int32),
        pltpu.VMEM((ROWS_PW, CHUNK), jnp.int32),
        pltpu.VMEM((CHUNK, E), jnp.float32),
        pltpu.VMEM((CHUNK, E), jnp.float32),
        pltpu.VMEM((CHUNK, E), jnp.float32),
        pltpu.VMEM((CHUNK, E), jnp.float32),
        pltpu.VMEM((CHUNK,), jnp.int32),
        pltpu.VMEM((CHUNK,), jnp.int32),
        pltpu.VMEM((CHUNK,), jnp.int32),
        pltpu.VMEM((CHUNK,), jnp.int32),
        pltpu.VMEM((4, 16), jnp.float32),
        pltpu.SemaphoreType.DMA,
        pltpu.SemaphoreType.DMA,
    ],
    compiler_params=pltpu.CompilerParams(needs_layout_passes=False,
                                         use_tc_tiling_on_sc=False),
)(_sc_body)


def kernel(embedding, instance_mask):
    emb3 = embedding.reshape(B, E, HW)
    embt = _transpose(emb3).reshape(BHW, E)
    labels = instance_mask.reshape(BHW)
    idx1, idx2 = _build_indices()
    parts = _sc_kernel(embt, labels, idx1, idx2)
    g = parts.reshape(B, NW // B, 4, 16).sum(axis=(1, 3))
    pos_s, pos_c, neg_s, neg_c = g[:, 0], g[:, 1], g[:, 2], g[:, 3]
    pos = jnp.where(pos_c > 0, pos_s / jnp.maximum(pos_c, 1.0), 0.0)
    neg = jnp.where(neg_c > 0, neg_s / jnp.maximum(neg_c, 1.0), 0.0)
    total_pos = jnp.sum(pos) / B
    total_neg = jnp.sum(neg) / B
    total = total_pos + total_neg
    return (total, total_pos, total_neg)


# speedup vs baseline: 2.2292x; 2.2292x over previous
import functools

import jax
import jax.numpy as jnp
from jax import lax
from jax.experimental import pallas as pl
from jax.experimental.pallas import tpu as pltpu
from jax.experimental.pallas import tpu_sc as plsc

MARGIN = 1.0
N_SAMPLES = 65536
B = 4
E = 32
H = 512
W = 512
HW = H * W
BHW = B * HW
PACK = 128 // E
NROWS = BHW // PACK

NC = 2
NS = 16
NW = NC * NS
PAIRS = B * N_SAMPLES
PPW = PAIRS // NW
CHUNK = 128
ROWS_PW = PPW // CHUNK
NT = ROWS_PW // 2

HB = 8


def _build_indices():
    i1, i2 = [], []
    for b in range(B):
        kb = jax.random.fold_in(jax.random.key(42), b)
        ka, kc = jax.random.split(kb)
        i1.append(jax.random.randint(ka, (N_SAMPLES,), 0, HW) + b * HW)
        i2.append(jax.random.randint(kc, (N_SAMPLES,), 0, HW) + b * HW)
    g1 = jnp.concatenate(i1).astype(jnp.int32).reshape(PAIRS // CHUNK, CHUNK)
    g2 = jnp.concatenate(i2).astype(jnp.int32).reshape(PAIRS // CHUNK, CHUNK)
    q1 = ((g1 >> 18) << 16) + (g1 & 0xFFFF)
    q2 = ((g2 >> 18) << 16) + (g2 & 0xFFFF)
    return q1, q2, g1, g2


def _tr_body(x0, x1, x2, x3, o_ref):
    for q, x in enumerate((x0, x1, x2, x3)):
        for hh in range(HB):
            o_ref[pl.ds(hh * W, W), pl.ds(q * E, E)] = x[0, :, hh, :].T


def _transpose(emb4):
    hq = H // PACK // HB
    specs = [
        pl.BlockSpec((1, E, HB, W),
                     lambda b, j, q=q: (b, 0, q * hq + j, 0))
        for q in range(PACK)
    ]
    return pl.pallas_call(
        _tr_body,
        grid=(B, hq),
        in_specs=specs,
        out_specs=pl.BlockSpec((HB * W, 128), lambda b, j: (b * hq + j, 0)),
        out_shape=jax.ShapeDtypeStruct((NROWS, 128), jnp.float32),
    )(emb4, emb4, emb4, emb4)


def _chunk_contrib(jj, g1_v, g2_v, a_ref, b_ref, l1_ref, l2_ref, accs):
    pos_s, pos_c, neg_s, neg_c = accs
    lane = lax.iota(jnp.int32, 16)
    one = jnp.float32(1.0)
    zero = jnp.float32(0.0)
    for k in range(CHUNK // 16):
        rows = lane + (k * 16)
        gv1 = g1_v[jj, pl.ds(k * 16, 16)]
        gv2 = g2_v[jj, pl.ds(k * 16, 16)]
        col1 = ((gv1 >> 16) & 3) << 5
        col2 = ((gv2 >> 16) & 3) << 5
        d2 = jnp.zeros(16, jnp.float32)
        for c in range(E):
            av = plsc.load_gather(a_ref, [rows, col1 + c])
            bv = plsc.load_gather(b_ref, [rows, col2 + c])
            d = av - bv
            d2 = d2 + d * d
        l1 = l1_ref[pl.ds(k * 16, 16)]
        l2 = l2_ref[pl.ds(k * 16, 16)]
        same = (l1 == l2) & (l1 != 0)
        diff = (l1 != l2) & (l1 != 0) & (l2 != 0)
        d2e = d2 + jnp.float32(1e-12)
        ir = jnp.int32(0x5F3759DF) - (plsc.bitcast(d2e, jnp.int32) >> 1)
        r = plsc.bitcast(ir, jnp.float32)
        r = r * (jnp.float32(1.5) - jnp.float32(0.5) * d2e * r * r)
        r = r * (jnp.float32(1.5) - jnp.float32(0.5) * d2e * r * r)
        dist = d2e * r
        hin = jnp.maximum(jnp.float32(MARGIN) - dist, zero)
        pos_s = pos_s + jnp.where(same, d2e, zero)
        pos_c = pos_c + jnp.where(same, one, zero)
        neg_s = neg_s + jnp.where(diff, hin * hin, zero)
        neg_c = neg_c + jnp.where(diff, one, zero)
    return pos_s, pos_c, neg_s, neg_c


def _sc_body(emb_hbm, lab_hbm, q1_hbm, q2_hbm, g1_hbm, g2_hbm, out_hbm,
             q1_v, q2_v, g1_v, g2_v, a0, a1, b0, b1, l10, l11, l20, l21,
             accv, sem0, sem1):
    wid = lax.axis_index("s") * NC + lax.axis_index("c")
    base = wid * ROWS_PW
    pltpu.sync_copy(q1_hbm.at[pl.ds(base, ROWS_PW)], q1_v)
    pltpu.sync_copy(q2_hbm.at[pl.ds(base, ROWS_PW)], q2_v)
    pltpu.sync_copy(g1_hbm.at[pl.ds(base, ROWS_PW)], g1_v)
    pltpu.sync_copy(g2_hbm.at[pl.ds(base, ROWS_PW)], g2_v)

    bufs = ((a0, b0, l10, l20, sem0), (a1, b1, l11, l21, sem1))

    def _issue(jj, a, b, l1, l2, sem):
        pltpu.async_copy(emb_hbm.at[q1_v.at[jj]], a, sem)
        pltpu.async_copy(emb_hbm.at[q2_v.at[jj]], b, sem)
        pltpu.async_copy(lab_hbm.at[g1_v.at[jj]], l1, sem)
        pltpu.async_copy(lab_hbm.at[g2_v.at[jj]], l2, sem)

    def _drain(a, b, l1, l2, sem):
        pltpu.make_async_copy(emb_hbm.at[q1_v.at[0]], a, sem).wait()
        pltpu.make_async_copy(emb_hbm.at[q2_v.at[0]], b, sem).wait()
        pltpu.make_async_copy(lab_hbm.at[g1_v.at[0]], l1, sem).wait()
        pltpu.make_async_copy(lab_hbm.at[g2_v.at[0]], l2, sem).wait()

    _issue(jnp.int32(0), *bufs[0])

    def body(t, accs):
        j0 = 2 * t
        _issue(j0 + 1, *bufs[1])
        _drain(*bufs[0])
        accs = _chunk_contrib(j0, g1_v, g2_v, bufs[0][0], bufs[0][1],
                              bufs[0][2], bufs[0][3], accs)

        @pl.when(t < NT - 1)
        def _():
            _issue(j0 + 2, *bufs[0])

        _drain(*bufs[1])
        accs = _chunk_contrib(j0 + 1, g1_v, g2_v, bufs[1][0], bufs[1][1],
                              bufs[1][2], bufs[1][3], accs)
        return accs

    z = jnp.zeros(16, jnp.float32)
    pos_s, pos_c, neg_s, neg_c = lax.fori_loop(0, NT, body, (z, z, z, z))
    accv[pl.ds(0, 16)] = pos_s
    accv[pl.ds(16, 16)] = pos_c
    accv[pl.ds(32, 16)] = neg_s
    accv[pl.ds(48, 16)] = neg_c
    pltpu.sync_copy(accv, out_hbm.at[pl.ds(wid * 64, 64)])


_sc_kernel = functools.partial(
    pl.kernel,
    out_type=jax.ShapeDtypeStruct((NW * 64,), jnp.float32),
    mesh=plsc.VectorSubcoreMesh(core_axis_name="c", subcore_axis_name="s",
                                num_cores=NC, num_subcores=NS),
    scratch_types=[
        pltpu.VMEM((ROWS_PW, CHUNK), jnp.int32),
        pltpu.VMEM((ROWS_PW, CHUNK), jnp.int32),
        pltpu.VMEM((ROWS_PW, CHUNK), jnp.int32),
        pltpu.VMEM((ROWS_PW, CHUNK), jnp.int32),
        pltpu.VMEM((CHUNK, 128), jnp.float32),
        pltpu.VMEM((CHUNK, 128), jnp.float32),
        pltpu.VMEM((CHUNK, 128), jnp.float32),
        pltpu.VMEM((CHUNK, 128), jnp.float32),
        pltpu.VMEM((CHUNK,), jnp.int32),
        pltpu.VMEM((CHUNK,), jnp.int32),
        pltpu.VMEM((CHUNK,), jnp.int32),
        pltpu.VMEM((CHUNK,), jnp.int32),
        pltpu.VMEM((64,), jnp.float32),
        pltpu.SemaphoreType.DMA,
        pltpu.SemaphoreType.DMA,
    ],
    compiler_params=pltpu.CompilerParams(needs_layout_passes=False,
                                         use_tc_tiling_on_sc=False),
)(_sc_body)


def kernel(embedding, instance_mask):
    embp = _transpose(embedding)
    labels = instance_mask.reshape(BHW)
    q1, q2, g1, g2 = _build_indices()
    parts = _sc_kernel(embp, labels, q1, q2, g1, g2)
    g = parts.reshape(B, NW // B, 4, 16).sum(axis=(1, 3))
    pos_s, pos_c, neg_s, neg_c = g[:, 0], g[:, 1], g[:, 2], g[:, 3]
    pos = jnp.where(pos_c > 0, pos_s / jnp.maximum(pos_c, 1.0), 0.0)
    neg = jnp.where(neg_c > 0, neg_s / jnp.maximum(neg_c, 1.0), 0.0)
    total_pos = jnp.sum(pos) / B
    total_neg = jnp.sum(neg) / B
    total = total_pos + total_neg
    return (total, total_pos, total_neg)
